# baseline (speedup 1.0000x reference)
import jax
import jax.numpy as jnp
from jax import lax
from jax.experimental import pallas as pl
from jax.experimental.pallas import tpu as pltpu
from jax.experimental.pallas import tpu_sc as plsc

BATCH = 16384
N_ROWS = 100000
K = 16
KT = 4
L = 16
NW = 32
BPW = BATCH // NW
NTILE = 782
SPAN = 25
TPW = 8
NTASK = 4
W = TPW * 128
CHUNK = 32


def _transpose(blkbuf, obkbuf, lanes, kdim, dyn0):
    GB = 8
    if kdim == K:
        def blk_body(gb, car):
            j0 = gb * GB
            for m in range(GB):
                col = jnp.full((L,), 0, jnp.int32) + (j0 + m)
                vec = plsc.load_gather(blkbuf, [lanes, col])
                obkbuf[gb, pl.ds(m * L, L)] = vec
            return car

        lax.fori_loop(0, W // GB + dyn0, blk_body, 0)
    else:
        la3 = lanes & 3
        l2v = lanes >> 2

        def blk_body(gb, car):
            j0 = gb * (GB * KT)
            for m in range(GB):
                col = l2v + (j0 + m * KT)
                vec = plsc.load_gather(blkbuf, [la3, col])
                obkbuf[gb, pl.ds(m * L, L)] = vec
            return car

        lax.fori_loop(0, W // (GB * KT) + dyn0, blk_body, 0)


def _retile_table(tabT, scr, bA, bB, oA, oB, sinA, sinB, soutA, soutB,
                  start, lanes, kdim, dyn0):
    orows = W * kdim // 128
    rpt = kdim

    def base(t):
        return start + jnp.minimum(TPW * t, SPAN - TPW)

    def issue_in(t, buf, sem):
        pltpu.async_copy(tabT.at[:, pl.ds(base(t) * 128, W)], buf, sem)

    def issue_out(t, buf, sem):
        pltpu.async_copy(buf, scr.at[pl.ds(base(t) * rpt, orows), :], sem)

    def w_in(buf, sem):
        pltpu.make_async_copy(tabT.at[:, pl.ds(0, W)], buf, sem).wait()

    def w_out(buf, sem):
        pltpu.make_async_copy(buf, scr.at[pl.ds(0, orows), :], sem).wait()

    issue_in(0, bA, sinA)
    npairs = NTASK // 2

    def body2(jj, car):
        t0 = 2 * jj
        issue_in(t0 + 1, bB, sinB)
        w_in(bA, sinA)

        @pl.when(jj > 0)
        def _():
            w_out(oA, soutA)

        _transpose(bA, oA, lanes, kdim, dyn0)
        issue_out(t0, oA, soutA)

        @pl.when(jj < npairs - 1)
        def _():
            issue_in(t0 + 2, bA, sinA)

        w_in(bB, sinB)

        @pl.when(jj > 0)
        def _():
            w_out(oB, soutB)

        _transpose(bB, oB, lanes, kdim, dyn0)
        issue_out(t0 + 1, oB, soutB)
        return car

    lax.fori_loop(0, npairs + dyn0, body2, 0)
    w_out(oA, soutA)
    w_out(oB, soutB)


def _mf_kernel(uid_h, iid_h, rid_h, oid_h,
               uT, iT, oT, utT, tT, bu_h, bi_h, bias_h,
               out_h,
               u_scr, i_scr, o_scr, ut_scr, t_scr,
               uid_v, iid_v, rid_v, oid_v,
               gu_v, gi_v, go_v, gut_v, gt_v,
               bA, bB, oA, oB, btA, btB, otA, otB,
               ub, ib, ob, utb, tbuf,
               bu_v, bi_v, bias_v, out_v,
               sem, sinA, sinB, soutA, soutB, bsem):
    nc = 2
    cid = lax.axis_index("c")
    sid = lax.axis_index("s")
    wid = sid * nc + cid
    base_g = wid * BPW
    sync = pltpu.sync_copy

    sync(uid_h.at[pl.ds(base_g, BPW)], uid_v)
    sync(iid_h.at[pl.ds(base_g, BPW)], iid_v)
    sync(rid_h.at[pl.ds(base_g, BPW)], rid_v)
    sync(oid_h.at[pl.ds(base_g, BPW)], oid_v)
    sync(bias_h, bias_v)
    bias_copies = [
        pltpu.async_copy(bu_h.at[uid_v], bu_v, sem),
        pltpu.async_copy(bi_h.at[iid_v], bi_v, sem),
    ]

    lanes = lax.iota(jnp.int32, L)

    def derive(g, _):
        s = pl.ds(g * L, L)
        u = uid_v[s]
        gu_v[s] = u >> 3
        gut_v[s] = u >> 5
        gi_v[s] = iid_v[s] >> 3
        go_v[s] = oid_v[s] >> 3
        gt_v[s] = rid_v[s] >> 5
        return _

    lax.fori_loop(0, BPW // L + 0 * wid, derive, 0)

    start = jnp.minimum(wid * SPAN, NTILE - SPAN)
    dyn0 = 0 * wid
    _retile_table(uT, u_scr, bA, bB, oA, oB, sinA, sinB, soutA, soutB,
                  start, lanes, K, dyn0)
    _retile_table(iT, i_scr, bA, bB, oA, oB, sinA, sinB, soutA, soutB,
                  start, lanes, K, dyn0)
    _retile_table(oT, o_scr, bA, bB, oA, oB, sinA, sinB, soutA, soutB,
                  start, lanes, K, dyn0)
    _retile_table(utT, ut_scr, btA, btB, otA, otB, sinA, sinB, soutA, soutB,
                  start, lanes, KT, dyn0)
    _retile_table(tT, t_scr, btA, btB, otA, otB, sinA, sinB, soutA, soutB,
                  start, lanes, KT, dyn0)

    plsc.subcore_barrier()

    @pl.when(sid == 0)
    def _():
        pltpu.semaphore_signal(bsem, 1, core_index=1 - cid)
        pltpu.semaphore_wait(bsem, 1)

    plsc.subcore_barrier()

    for c in bias_copies:
        c.wait()
    bvec = bias_v[...]

    def chunk_body(ch, _c):
        cb = ch * CHUNK
        copies = [
            pltpu.async_copy(u_scr.at[gu_v.at[pl.ds(cb, CHUNK)]], ub, sem),
            pltpu.async_copy(i_scr.at[gi_v.at[pl.ds(cb, CHUNK)]], ib, sem),
            pltpu.async_copy(o_scr.at[go_v.at[pl.ds(cb, CHUNK)]], ob, sem),
            pltpu.async_copy(ut_scr.at[gut_v.at[pl.ds(cb, CHUNK)]], utb, sem),
            pltpu.async_copy(t_scr.at[gt_v.at[pl.ds(cb, CHUNK)]], tbuf, sem),
        ]
        for c in copies:
            c.wait()

        def group(g, _g):
            base = cb + g * L
            rows = lanes + g * L
            uidg = uid_v[pl.ds(base, L)]
            iidg = iid_v[pl.ds(base, L)]
            ridg = rid_v[pl.ds(base, L)]
            oidg = oid_v[pl.ds(base, L)]
            ucol = (uidg & 7) << 4
            icol = (iidg & 7) << 4
            ocol = (oidg & 7) << 4
            utcol = (uidg & 31) << 2
            tcol = (ridg & 31) << 2
            accs = [bvec, bu_v[pl.ds(base, L)], bi_v[pl.ds(base, L)],
                    jnp.zeros((L,), jnp.float32)]
            for k in range(K):
                uc = plsc.load_gather(ub, [rows, ucol + k])
                ic = plsc.load_gather(ib, [rows, icol + k])
                oc = plsc.load_gather(ob, [rows, ocol + k])
                accs[k % 4] = accs[k % 4] + uc * (ic + oc)
            for k in range(KT):
                utc = plsc.load_gather(utb, [rows, utcol + k])
                tc = plsc.load_gather(tbuf, [rows, tcol + k])
                accs[k] = accs[k] + utc * tc
            out_v[pl.ds(base, L)] = (accs[0] + accs[1]) + (accs[2] + accs[3])
            return _g

        lax.fori_loop(0, CHUNK // L + dyn0, group, 0)
        return _c

    lax.fori_loop(0, BPW // CHUNK + dyn0, chunk_body, 0)

    sync(out_v, out_h.at[pl.ds(base_g, BPW)])


@jax.jit
def kernel(train_x, user_W, item_W, occu_W, user_temp_W, temp_W,
           bias_user_W, bias_item_W, bias):
    uid = train_x[:, 0]
    iid = train_x[:, 1]
    rid = train_x[:, 2]
    oid = train_x[:, 3]
    uT = user_W.T
    iT = item_W.T
    oT = occu_W.T
    utT = user_temp_W.T
    tT = temp_W.T
    bu = bias_user_W.reshape(-1)
    bi = bias_item_W.reshape(-1)
    bias16 = jnp.broadcast_to(bias, (L,))

    mesh = plsc.VectorSubcoreMesh(core_axis_name="c", subcore_axis_name="s")
    f = pl.kernel(
        _mf_kernel,
        mesh=mesh,
        out_type=jax.ShapeDtypeStruct((BATCH,), jnp.float32),
        scratch_types=[
            pltpu.HBM((NTILE * K, 128), jnp.float32),
            pltpu.HBM((NTILE * K, 128), jnp.float32),
            pltpu.HBM((NTILE * K, 128), jnp.float32),
            pltpu.HBM((NTILE * KT, 128), jnp.float32),
            pltpu.HBM((NTILE * KT, 128), jnp.float32),
            pltpu.VMEM((BPW,), jnp.int32),
            pltpu.VMEM((BPW,), jnp.int32),
            pltpu.VMEM((BPW,), jnp.int32),
            pltpu.VMEM((BPW,), jnp.int32),
            pltpu.VMEM((BPW,), jnp.int32),
            pltpu.VMEM((BPW,), jnp.int32),
            pltpu.VMEM((BPW,), jnp.int32),
            pltpu.VMEM((BPW,), jnp.int32),
            pltpu.VMEM((BPW,), jnp.int32),
            pltpu.VMEM((K, W), jnp.float32),
            pltpu.VMEM((K, W), jnp.float32),
            pltpu.VMEM((W * K // 128, 128), jnp.float32),
            pltpu.VMEM((W * K // 128, 128), jnp.float32),
            pltpu.VMEM((KT, W), jnp.float32),
            pltpu.VMEM((KT, W), jnp.float32),
            pltpu.VMEM((W * KT // 128, 128), jnp.float32),
            pltpu.VMEM((W * KT // 128, 128), jnp.float32),
            pltpu.VMEM((CHUNK, 128), jnp.float32),
            pltpu.VMEM((CHUNK, 128), jnp.float32),
            pltpu.VMEM((CHUNK, 128), jnp.float32),
            pltpu.VMEM((CHUNK, 128), jnp.float32),
            pltpu.VMEM((CHUNK, 128), jnp.float32),
            pltpu.VMEM((BPW,), jnp.float32),
            pltpu.VMEM((BPW,), jnp.float32),
            pltpu.VMEM((L,), jnp.float32),
            pltpu.VMEM((BPW,), jnp.float32),
            pltpu.SemaphoreType.DMA,
            pltpu.SemaphoreType.DMA,
            pltpu.SemaphoreType.DMA,
            pltpu.SemaphoreType.DMA,
            pltpu.SemaphoreType.DMA,
            pltpu.SemaphoreType.REGULAR,
        ],
        compiler_params=pltpu.CompilerParams(needs_layout_passes=False),
    )
    return f(uid, iid, rid, oid, uT, iT, oT, utT, tT, bu, bi, bias16)


# speedup vs baseline: 1.5434x; 1.5434x over previous
import jax
import jax.numpy as jnp
from jax import lax
from jax.experimental import pallas as pl
from jax.experimental.pallas import tpu as pltpu
from jax.experimental.pallas import tpu_sc as plsc

BATCH = 16384
N_ROWS = 100000
K = 16
KT = 4
L = 16
NW = 32
BPW = BATCH // NW
NTILE = 782
SPAN = 25
TPW = 8
NTASK = 4
W = TPW * 128
CHUNK = 32


def _transpose(blkbuf, obkbuf, lanes, kdim, dyn0):
    GB = 8
    if kdim == K:
        l3 = lanes >> 3
        kv = [jnp.full((L,), k, jnp.int32) for k in range(K)]
        pc = [((lanes & 7) << 4) + k for k in range(K)]
        sv = [lanes + j * L for j in range(GB)]
        rv = [l3 + 2 * j for j in range(GB)]

        def blk_body(gb, car):
            for j in range(GB):
                col = sv[j] + gb * (GB * L)
                row = rv[j] + gb * (2 * GB)
                for k in range(K):
                    vec = plsc.load_gather(blkbuf, [kv[k], col])
                    plsc.store_scatter(obkbuf, [row, pc[k]], vec)
            return car

        lax.fori_loop(0, W // L // GB + dyn0, blk_body, 0)
    else:
        l2 = lanes << 2
        pr = l2 >> 7
        kv = [jnp.full((L,), k, jnp.int32) for k in range(KT)]
        cv = [[l2 + (par * 64 + k) for k in range(KT)] for par in range(2)]
        sv = [lanes + j * L for j in range(GB)]
        rv = [pr + (j >> 1) for j in range(GB)]

        def blk_body(gb, car):
            for j in range(GB):
                col = sv[j] + gb * (GB * L)
                row = rv[j] + gb * (GB // 2)
                for k in range(KT):
                    vec = plsc.load_gather(blkbuf, [kv[k], col])
                    plsc.store_scatter(obkbuf, [row, cv[j & 1][k]], vec)
            return car

        lax.fori_loop(0, W // L // GB + dyn0, blk_body, 0)


def _retile_table(tabT, scr, bA, bB, oA, oB, sinA, sinB, soutA, soutB,
                  start, lanes, kdim, dyn0):
    orows = W * kdim // 128
    rpt = kdim

    def base(t):
        return start + jnp.minimum(TPW * t, SPAN - TPW)

    def issue_in(t, buf, sem):
        pltpu.async_copy(tabT.at[:, pl.ds(base(t) * 128, W)], buf, sem)

    def issue_out(t, buf, sem):
        pltpu.async_copy(buf, scr.at[pl.ds(base(t) * rpt, orows), :], sem)

    def w_in(buf, sem):
        pltpu.make_async_copy(tabT.at[:, pl.ds(0, W)], buf, sem).wait()

    def w_out(buf, sem):
        pltpu.make_async_copy(buf, scr.at[pl.ds(0, orows), :], sem).wait()

    issue_in(0, bA, sinA)
    npairs = NTASK // 2

    def body2(jj, car):
        t0 = 2 * jj
        issue_in(t0 + 1, bB, sinB)
        w_in(bA, sinA)

        @pl.when(jj > 0)
        def _():
            w_out(oA, soutA)

        _transpose(bA, oA, lanes, kdim, dyn0)
        issue_out(t0, oA, soutA)

        @pl.when(jj < npairs - 1)
        def _():
            issue_in(t0 + 2, bA, sinA)

        w_in(bB, sinB)

        @pl.when(jj > 0)
        def _():
            w_out(oB, soutB)

        _transpose(bB, oB, lanes, kdim, dyn0)
        issue_out(t0 + 1, oB, soutB)
        return car

    lax.fori_loop(0, npairs + dyn0, body2, 0)
    w_out(oA, soutA)
    w_out(oB, soutB)


def _mf_kernel(uid_h, iid_h, rid_h, oid_h,
               uT, iT, oT, utT, tT, bu_h, bi_h, bias_h,
               out_h,
               u_scr, i_scr, o_scr, ut_scr, t_scr,
               uid_v, iid_v, rid_v, oid_v,
               gu_v, gi_v, go_v, gut_v, gt_v,
               bA, bB, oA, oB, btA, btB, otA, otB,
               ub, ib, ob, utb, tbuf,
               bu_v, bi_v, bias_v, out_v,
               sem, sinA, sinB, soutA, soutB, bsem):
    nc = 2
    cid = lax.axis_index("c")
    sid = lax.axis_index("s")
    wid = sid * nc + cid
    base_g = wid * BPW
    sync = pltpu.sync_copy

    sync(uid_h.at[pl.ds(base_g, BPW)], uid_v)
    sync(iid_h.at[pl.ds(base_g, BPW)], iid_v)
    sync(rid_h.at[pl.ds(base_g, BPW)], rid_v)
    sync(oid_h.at[pl.ds(base_g, BPW)], oid_v)
    sync(bias_h, bias_v)
    bias_copies = [
        pltpu.async_copy(bu_h.at[uid_v], bu_v, sem),
        pltpu.async_copy(bi_h.at[iid_v], bi_v, sem),
    ]

    lanes = lax.iota(jnp.int32, L)

    def derive(g, _):
        s = pl.ds(g * L, L)
        u = uid_v[s]
        gu_v[s] = u >> 3
        gut_v[s] = u >> 5
        gi_v[s] = iid_v[s] >> 3
        go_v[s] = oid_v[s] >> 3
        gt_v[s] = rid_v[s] >> 5
        return _

    lax.fori_loop(0, BPW // L + 0 * wid, derive, 0)

    start = jnp.minimum(wid * SPAN, NTILE - SPAN)
    dyn0 = 0 * wid
    _retile_table(uT, u_scr, bA, bB, oA, oB, sinA, sinB, soutA, soutB,
                  start, lanes, K, dyn0)
    _retile_table(iT, i_scr, bA, bB, oA, oB, sinA, sinB, soutA, soutB,
                  start, lanes, K, dyn0)
    _retile_table(oT, o_scr, bA, bB, oA, oB, sinA, sinB, soutA, soutB,
                  start, lanes, K, dyn0)
    _retile_table(utT, ut_scr, btA, btB, otA, otB, sinA, sinB, soutA, soutB,
                  start, lanes, KT, dyn0)
    _retile_table(tT, t_scr, btA, btB, otA, otB, sinA, sinB, soutA, soutB,
                  start, lanes, KT, dyn0)

    plsc.subcore_barrier()

    @pl.when(sid == 0)
    def _():
        pltpu.semaphore_signal(bsem, 1, core_index=1 - cid)
        pltpu.semaphore_wait(bsem, 1)

    plsc.subcore_barrier()

    for c in bias_copies:
        c.wait()
    bvec = bias_v[...]

    def chunk_body(ch, _c):
        cb = ch * CHUNK
        copies = [
            pltpu.async_copy(u_scr.at[gu_v.at[pl.ds(cb, CHUNK)]], ub, sem),
            pltpu.async_copy(i_scr.at[gi_v.at[pl.ds(cb, CHUNK)]], ib, sem),
            pltpu.async_copy(o_scr.at[go_v.at[pl.ds(cb, CHUNK)]], ob, sem),
            pltpu.async_copy(ut_scr.at[gut_v.at[pl.ds(cb, CHUNK)]], utb, sem),
            pltpu.async_copy(t_scr.at[gt_v.at[pl.ds(cb, CHUNK)]], tbuf, sem),
        ]
        for c in copies:
            c.wait()

        def group(g, _g):
            base = cb + g * L
            rows = lanes + g * L
            uidg = uid_v[pl.ds(base, L)]
            iidg = iid_v[pl.ds(base, L)]
            ridg = rid_v[pl.ds(base, L)]
            oidg = oid_v[pl.ds(base, L)]
            ucol = (uidg & 7) << 4
            icol = (iidg & 7) << 4
            ocol = (oidg & 7) << 4
            utcol = (uidg & 31) << 2
            tcol = (ridg & 31) << 2
            accs = [bvec, bu_v[pl.ds(base, L)], bi_v[pl.ds(base, L)],
                    jnp.zeros((L,), jnp.float32)]
            for k in range(K):
                uc = plsc.load_gather(ub, [rows, ucol + k])
                ic = plsc.load_gather(ib, [rows, icol + k])
                oc = plsc.load_gather(ob, [rows, ocol + k])
                accs[k % 4] = accs[k % 4] + uc * (ic + oc)
            for k in range(KT):
                utc = plsc.load_gather(utb, [rows, utcol + k])
                tc = plsc.load_gather(tbuf, [rows, tcol + k])
                accs[k] = accs[k] + utc * tc
            out_v[pl.ds(base, L)] = (accs[0] + accs[1]) + (accs[2] + accs[3])
            return _g

        lax.fori_loop(0, CHUNK // L + dyn0, group, 0)
        return _c

    lax.fori_loop(0, BPW // CHUNK + dyn0, chunk_body, 0)

    sync(out_v, out_h.at[pl.ds(base_g, BPW)])


@jax.jit
def kernel(train_x, user_W, item_W, occu_W, user_temp_W, temp_W,
           bias_user_W, bias_item_W, bias):
    uid = train_x[:, 0]
    iid = train_x[:, 1]
    rid = train_x[:, 2]
    oid = train_x[:, 3]
    uT = user_W.T
    iT = item_W.T
    oT = occu_W.T
    utT = user_temp_W.T
    tT = temp_W.T
    bu = bias_user_W.reshape(-1)
    bi = bias_item_W.reshape(-1)
    bias16 = jnp.broadcast_to(bias, (L,))

    mesh = plsc.VectorSubcoreMesh(core_axis_name="c", subcore_axis_name="s")
    f = pl.kernel(
        _mf_kernel,
        mesh=mesh,
        out_type=jax.ShapeDtypeStruct((BATCH,), jnp.float32),
        scratch_types=[
            pltpu.HBM((NTILE * K, 128), jnp.float32),
            pltpu.HBM((NTILE * K, 128), jnp.float32),
            pltpu.HBM((NTILE * K, 128), jnp.float32),
            pltpu.HBM((NTILE * KT, 128), jnp.float32),
            pltpu.HBM((NTILE * KT, 128), jnp.float32),
            pltpu.VMEM((BPW,), jnp.int32),
            pltpu.VMEM((BPW,), jnp.int32),
            pltpu.VMEM((BPW,), jnp.int32),
            pltpu.VMEM((BPW,), jnp.int32),
            pltpu.VMEM((BPW,), jnp.int32),
            pltpu.VMEM((BPW,), jnp.int32),
            pltpu.VMEM((BPW,), jnp.int32),
            pltpu.VMEM((BPW,), jnp.int32),
            pltpu.VMEM((BPW,), jnp.int32),
            pltpu.VMEM((K, W), jnp.float32),
            pltpu.VMEM((K, W), jnp.float32),
            pltpu.VMEM((W * K // 128, 128), jnp.float32),
            pltpu.VMEM((W * K // 128, 128), jnp.float32),
            pltpu.VMEM((KT, W), jnp.float32),
            pltpu.VMEM((KT, W), jnp.float32),
            pltpu.VMEM((W * KT // 128, 128), jnp.float32),
            pltpu.VMEM((W * KT // 128, 128), jnp.float32),
            pltpu.VMEM((CHUNK, 128), jnp.float32),
            pltpu.VMEM((CHUNK, 128), jnp.float32),
            pltpu.VMEM((CHUNK, 128), jnp.float32),
            pltpu.VMEM((CHUNK, 128), jnp.float32),
            pltpu.VMEM((CHUNK, 128), jnp.float32),
            pltpu.VMEM((BPW,), jnp.float32),
            pltpu.VMEM((BPW,), jnp.float32),
            pltpu.VMEM((L,), jnp.float32),
            pltpu.VMEM((BPW,), jnp.float32),
            pltpu.SemaphoreType.DMA,
            pltpu.SemaphoreType.DMA,
            pltpu.SemaphoreType.DMA,
            pltpu.SemaphoreType.DMA,
            pltpu.SemaphoreType.DMA,
            pltpu.SemaphoreType.REGULAR,
        ],
        compiler_params=pltpu.CompilerParams(needs_layout_passes=False),
    )
    return f(uid, iid, rid, oid, uT, iT, oT, utT, tT, bu, bi, bias16)
